# baseline (speedup 1.0000x reference)
import functools

import jax
import jax.numpy as jnp
from jax.experimental import pallas as pl
from jax.experimental.pallas import tpu as pltpu


def _body(num_blocks,
          adj_ref, cx_ref, gx_ref,
          wcs_ref, bcs_ref, wcn_ref, bcn_ref,
          wgs_ref, bgs_ref, wgn_ref, bgn_ref,
          cell_out_ref, gene_out_ref,
          gene_acc_ref):
    i = pl.program_id(0)
    a = adj_ref[...]
    cx = cx_ref[...]
    gx = gx_ref[...]

    cn = jnp.dot(a, gx, preferred_element_type=jnp.float32)
    cell_out_ref[...] = jnp.maximum(
        jnp.dot(cx, wcs_ref[...].T, preferred_element_type=jnp.float32)
        + bcs_ref[...]
        + jnp.dot(cn, wcn_ref[...].T, preferred_element_type=jnp.float32)
        + bcn_ref[...],
        0.0)

    gn_part = jax.lax.dot_general(
        a, cx, (((0,), (0,)), ((), ())),
        preferred_element_type=jnp.float32)

    @pl.when(i == 0)
    def _init():
        gene_acc_ref[...] = gn_part

    @pl.when(i > 0)
    def _accum():
        gene_acc_ref[...] += gn_part

    @pl.when(i == num_blocks - 1)
    def _finish():
        gn = gene_acc_ref[...]
        gene_out_ref[...] = jnp.maximum(
            jnp.dot(gx, wgs_ref[...].T, preferred_element_type=jnp.float32)
            + bgs_ref[...]
            + jnp.dot(gn, wgn_ref[...].T, preferred_element_type=jnp.float32)
            + bgn_ref[...],
            0.0)


def kernel(cell_x, gene_x, adjacency,
           W_cell_self, b_cell_self, W_cell_neigh, b_cell_neigh,
           W_gene_self, b_gene_self, W_gene_neigh, b_gene_neigh):
    N_c, d = cell_x.shape
    N_g = gene_x.shape[0]

    block = 1000
    assert N_c % block == 0
    num_blocks = N_c // block

    b_cell_self = b_cell_self.reshape(1, d)
    b_cell_neigh = b_cell_neigh.reshape(1, d)
    b_gene_self = b_gene_self.reshape(1, d)
    b_gene_neigh = b_gene_neigh.reshape(1, d)

    full = lambda shape: pl.BlockSpec(shape, lambda i: (0, 0))

    cell_out, gene_out = pl.pallas_call(
        functools.partial(_body, num_blocks),
        grid=(num_blocks,),
        in_specs=[
            pl.BlockSpec((block, N_g), lambda i: (i, 0)),
            pl.BlockSpec((block, d), lambda i: (i, 0)),
            full((N_g, d)),
            full((d, d)), full((1, d)),
            full((d, d)), full((1, d)),
            full((d, d)), full((1, d)),
            full((d, d)), full((1, d)),
        ],
        out_specs=[
            pl.BlockSpec((block, d), lambda i: (i, 0)),
            full((N_g, d)),
        ],
        out_shape=[
            jax.ShapeDtypeStruct((N_c, d), jnp.float32),
            jax.ShapeDtypeStruct((N_g, d), jnp.float32),
        ],
        scratch_shapes=[pltpu.VMEM((N_g, d), jnp.float32)],
        compiler_params=pltpu.CompilerParams(
            dimension_semantics=("arbitrary",),
        ),
    )(adjacency, cell_x, gene_x,
      W_cell_self, b_cell_self, W_cell_neigh, b_cell_neigh,
      W_gene_self, b_gene_self, W_gene_neigh, b_gene_neigh)

    return (cell_out, gene_out)


# speedup vs baseline: 1.0575x; 1.0575x over previous
import functools

import jax
import jax.numpy as jnp
from jax.experimental import pallas as pl
from jax.experimental.pallas import tpu as pltpu


def _body(num_blocks,
          adj_ref, cx_ref, cxt_ref, gx_ref,
          wcs_ref, wcn_ref, bc_ref,
          wgs_ref, wgn_ref, bg_ref,
          cell_out_ref, gene_out_ref,
          acc_ref):
    i = pl.program_id(0)
    a16 = adj_ref[...].astype(jnp.bfloat16)
    gx = gx_ref[...]

    cn = jnp.dot(a16, gx.astype(jnp.bfloat16),
                 preferred_element_type=jnp.float32)
    cx = cx_ref[...]
    cell_out_ref[...] = jnp.maximum(
        jnp.dot(cx, wcs_ref[...], preferred_element_type=jnp.float32)
        + jnp.dot(cn, wcn_ref[...], preferred_element_type=jnp.float32)
        + bc_ref[...],
        0.0)

    part = jax.lax.dot_general(
        cxt_ref[0].astype(jnp.bfloat16), a16,
        (((1,), (0,)), ((), ())),
        preferred_element_type=jnp.float32)

    @pl.when(i == 0)
    def _init():
        acc_ref[...] = part

    @pl.when(i > 0)
    def _accum():
        acc_ref[...] += part

    @pl.when(i == num_blocks - 1)
    def _finish():
        gn_t = acc_ref[...]
        gene_out_ref[...] = jnp.maximum(
            jnp.dot(gx, wgs_ref[...], preferred_element_type=jnp.float32)
            + jax.lax.dot_general(
                gn_t, wgn_ref[...], (((0,), (0,)), ((), ())),
                preferred_element_type=jnp.float32)
            + bg_ref[...],
            0.0)


def kernel(cell_x, gene_x, adjacency,
           W_cell_self, b_cell_self, W_cell_neigh, b_cell_neigh,
           W_gene_self, b_gene_self, W_gene_neigh, b_gene_neigh):
    N_c, d = cell_x.shape
    N_g = gene_x.shape[0]

    block = 1000
    assert N_c % block == 0
    num_blocks = N_c // block

    cell_x_t = cell_x.reshape(num_blocks, block, d).transpose(0, 2, 1)
    wcs = W_cell_self.T
    wcn = W_cell_neigh.T
    wgs = W_gene_self.T
    wgn = W_gene_neigh.T
    bc = (b_cell_self + b_cell_neigh).reshape(1, d)
    bg = (b_gene_self + b_gene_neigh).reshape(1, d)

    full = lambda shape: pl.BlockSpec(shape, lambda i: (0, 0))

    cell_out, gene_out = pl.pallas_call(
        functools.partial(_body, num_blocks),
        grid=(num_blocks,),
        in_specs=[
            pl.BlockSpec((block, N_g), lambda i: (i, 0)),
            pl.BlockSpec((block, d), lambda i: (i, 0)),
            pl.BlockSpec((1, d, block), lambda i: (i, 0, 0)),
            full((N_g, d)),
            full((d, d)), full((d, d)), full((1, d)),
            full((d, d)), full((d, d)), full((1, d)),
        ],
        out_specs=[
            pl.BlockSpec((block, d), lambda i: (i, 0)),
            full((N_g, d)),
        ],
        out_shape=[
            jax.ShapeDtypeStruct((N_c, d), jnp.float32),
            jax.ShapeDtypeStruct((N_g, d), jnp.float32),
        ],
        scratch_shapes=[pltpu.VMEM((d, N_g), jnp.float32)],
        compiler_params=pltpu.CompilerParams(
            dimension_semantics=("arbitrary",),
        ),
    )(adjacency, cell_x, cell_x_t, gene_x,
      wcs, wcn, bc, wgs, wgn, bg)

    return (cell_out, gene_out)


# speedup vs baseline: 1.1113x; 1.0509x over previous
import functools

import jax
import jax.numpy as jnp
from jax.experimental import pallas as pl
from jax.experimental.pallas import tpu as pltpu

_STREAMS = 4
_BLOCK = 200


def _body(num_blocks, *refs):
    adj_refs = refs[:_STREAMS]
    cx_refs = refs[_STREAMS:2 * _STREAMS]
    cxt_refs = refs[2 * _STREAMS:3 * _STREAMS]
    (gx_ref, wcs_ref, wcn_ref, bc_ref, wgs_ref, wgn_ref, bg_ref) = \
        refs[3 * _STREAMS:3 * _STREAMS + 7]
    cell_out_refs = refs[3 * _STREAMS + 7:4 * _STREAMS + 7]
    gene_out_ref = refs[4 * _STREAMS + 7]
    acc_ref = refs[4 * _STREAMS + 8]

    i = pl.program_id(0)
    gx = gx_ref[...]
    gx16 = gx.astype(jnp.bfloat16)
    wcs = wcs_ref[...]
    wcn = wcn_ref[...]
    bc = bc_ref[...]

    acc_add = None
    for j in range(_STREAMS):
        a16 = adj_refs[j][...].astype(jnp.bfloat16)
        cn = jnp.dot(a16, gx16, preferred_element_type=jnp.float32)
        cx = cx_refs[j][...]
        cell_out_refs[j][...] = jnp.maximum(
            jnp.dot(cx, wcs, preferred_element_type=jnp.float32)
            + jnp.dot(cn, wcn, preferred_element_type=jnp.float32)
            + bc,
            0.0)
        part = jax.lax.dot_general(
            cxt_refs[j][0].astype(jnp.bfloat16), a16,
            (((1,), (0,)), ((), ())),
            preferred_element_type=jnp.float32)
        acc_add = part if acc_add is None else acc_add + part

    @pl.when(i == 0)
    def _init():
        acc_ref[...] = acc_add

    @pl.when(i > 0)
    def _accum():
        acc_ref[...] += acc_add

    @pl.when(i == num_blocks - 1)
    def _finish():
        gene_out_ref[...] = jnp.maximum(
            jnp.dot(gx, wgs_ref[...], preferred_element_type=jnp.float32)
            + jax.lax.dot_general(
                acc_ref[...], wgn_ref[...], (((0,), (0,)), ((), ())),
                preferred_element_type=jnp.float32)
            + bg_ref[...],
            0.0)


def kernel(cell_x, gene_x, adjacency,
           W_cell_self, b_cell_self, W_cell_neigh, b_cell_neigh,
           W_gene_self, b_gene_self, W_gene_neigh, b_gene_neigh):
    N_c, d = cell_x.shape
    N_g = gene_x.shape[0]

    rows_per_stream = N_c // _STREAMS
    num_blocks = rows_per_stream // _BLOCK
    assert _STREAMS * num_blocks * _BLOCK == N_c

    total_blocks = _STREAMS * num_blocks
    cell_x_t = cell_x.reshape(total_blocks, _BLOCK, d).transpose(0, 2, 1)
    wcs = W_cell_self.T
    wcn = W_cell_neigh.T
    wgs = W_gene_self.T
    wgn = W_gene_neigh.T
    bc = (b_cell_self + b_cell_neigh).reshape(1, d)
    bg = (b_gene_self + b_gene_neigh).reshape(1, d)

    full = lambda shape: pl.BlockSpec(shape, lambda i: (0, 0))

    def row_map(j, i):
        return (j * num_blocks + i, 0)

    def cxt_map(j, i):
        return (j * num_blocks + i, 0, 0)

    adj_specs = [
        pl.BlockSpec((_BLOCK, N_g), functools.partial(row_map, j))
        for j in range(_STREAMS)
    ]
    cx_specs = [
        pl.BlockSpec((_BLOCK, d), functools.partial(row_map, j))
        for j in range(_STREAMS)
    ]
    cxt_specs = [
        pl.BlockSpec((1, d, _BLOCK), functools.partial(cxt_map, j))
        for j in range(_STREAMS)
    ]
    cell_out_specs = [
        pl.BlockSpec((_BLOCK, d), lambda i: (i, 0))
        for _ in range(_STREAMS)
    ]

    outs = pl.pallas_call(
        functools.partial(_body, num_blocks),
        grid=(num_blocks,),
        in_specs=[
            *adj_specs,
            *cx_specs,
            *cxt_specs,
            full((N_g, d)),
            full((d, d)), full((d, d)), full((1, d)),
            full((d, d)), full((d, d)), full((1, d)),
        ],
        out_specs=[
            *cell_out_specs,
            full((N_g, d)),
        ],
        out_shape=[
            *[jax.ShapeDtypeStruct((rows_per_stream, d), jnp.float32)
              for _ in range(_STREAMS)],
            jax.ShapeDtypeStruct((N_g, d), jnp.float32),
        ],
        scratch_shapes=[pltpu.VMEM((d, N_g), jnp.float32)],
        compiler_params=pltpu.CompilerParams(
            dimension_semantics=("arbitrary",),
        ),
    )(*([adjacency] * _STREAMS),
      *([cell_x] * _STREAMS),
      *([cell_x_t] * _STREAMS),
      gene_x, wcs, wcn, bc, wgs, wgn, bg)

    cell_out = jnp.concatenate(outs[:_STREAMS], axis=0)
    gene_out = outs[_STREAMS]
    return (cell_out, gene_out)


# speedup vs baseline: 2.7390x; 2.4646x over previous
import functools

import jax
import jax.numpy as jnp
from jax.experimental import pallas as pl
from jax.experimental.pallas import tpu as pltpu

_BLOCK_G = 80


def _body(num_blocks,
          at_ref, gxb_ref, cx_ref,
          wcs_ref, wcn_ref, bc_ref,
          wgs_ref, wgn_ref, bg_ref,
          gene_out_ref, cell_out_ref,
          acc_ref, cx16_ref):
    i = pl.program_id(0)

    @pl.when(i == 0)
    def _cache_cx():
        cx16_ref[...] = cx_ref[...].astype(jnp.bfloat16)

    at16 = at_ref[...].astype(jnp.bfloat16)
    gxb = gxb_ref[...]

    gn_blk = jnp.dot(at16, cx16_ref[...],
                     preferred_element_type=jnp.float32)
    gene_out_ref[...] = jnp.maximum(
        jnp.dot(gxb, wgs_ref[...], preferred_element_type=jnp.float32)
        + jnp.dot(gn_blk, wgn_ref[...], preferred_element_type=jnp.float32)
        + bg_ref[...],
        0.0)

    part = jax.lax.dot_general(
        gxb.astype(jnp.bfloat16), at16,
        (((0,), (0,)), ((), ())),
        preferred_element_type=jnp.float32)

    @pl.when(i == 0)
    def _init():
        acc_ref[...] = part

    @pl.when(i > 0)
    def _accum():
        acc_ref[...] += part

    @pl.when(i == num_blocks - 1)
    def _finish():
        cell_out_ref[...] = jnp.maximum(
            jnp.dot(cx_ref[...], wcs_ref[...],
                    preferred_element_type=jnp.float32)
            + jax.lax.dot_general(
                acc_ref[...], wcn_ref[...], (((0,), (0,)), ((), ())),
                preferred_element_type=jnp.float32)
            + bc_ref[...],
            0.0)


def kernel(cell_x, gene_x, adjacency,
           W_cell_self, b_cell_self, W_cell_neigh, b_cell_neigh,
           W_gene_self, b_gene_self, W_gene_neigh, b_gene_neigh):
    N_c, d = cell_x.shape
    N_g = gene_x.shape[0]

    num_blocks = N_g // _BLOCK_G
    assert num_blocks * _BLOCK_G == N_g

    at = adjacency.T
    wcs = W_cell_self.T
    wcn = W_cell_neigh.T
    wgs = W_gene_self.T
    wgn = W_gene_neigh.T
    bc = (b_cell_self + b_cell_neigh).reshape(1, d)
    bg = (b_gene_self + b_gene_neigh).reshape(1, d)

    full = lambda shape: pl.BlockSpec(shape, lambda i: (0, 0))

    gene_out, cell_out = pl.pallas_call(
        functools.partial(_body, num_blocks),
        grid=(num_blocks,),
        in_specs=[
            pl.BlockSpec((_BLOCK_G, N_c), lambda i: (i, 0)),
            pl.BlockSpec((_BLOCK_G, d), lambda i: (i, 0)),
            full((N_c, d)),
            full((d, d)), full((d, d)), full((1, d)),
            full((d, d)), full((d, d)), full((1, d)),
        ],
        out_specs=[
            pl.BlockSpec((_BLOCK_G, d), lambda i: (i, 0)),
            full((N_c, d)),
        ],
        out_shape=[
            jax.ShapeDtypeStruct((N_g, d), jnp.float32),
            jax.ShapeDtypeStruct((N_c, d), jnp.float32),
        ],
        scratch_shapes=[
            pltpu.VMEM((d, N_c), jnp.float32),
            pltpu.VMEM((N_c, d), jnp.bfloat16),
        ],
        compiler_params=pltpu.CompilerParams(
            dimension_semantics=("arbitrary",),
        ),
    )(at, gene_x, cell_x,
      wcs, wcn, bc, wgs, wgn, bg)

    return (cell_out, gene_out)


# speedup vs baseline: 3.2821x; 1.1983x over previous
import functools

import jax
import jax.numpy as jnp
from jax.experimental import pallas as pl
from jax.experimental.pallas import tpu as pltpu

_BLOCK_G = 160


def _body(num_blocks,
          at_ref, gxb_ref, cxt_ref,
          wcs_ref, wcn_ref, bc_ref,
          wgs_ref, wgn_ref, bg_ref,
          gene_out_ref, cot_ref,
          acc_ref, cxt16_ref):
    i = pl.program_id(0)

    @pl.when(i == 0)
    def _cache_cxt():
        cxt16_ref[...] = cxt_ref[...].astype(jnp.bfloat16)

    at16 = at_ref[...].astype(jnp.bfloat16)
    gxb = gxb_ref[...]

    gn_blk = jax.lax.dot_general(
        at16, cxt16_ref[...], (((1,), (1,)), ((), ())),
        preferred_element_type=jnp.float32)
    gene_out_ref[...] = jnp.maximum(
        jnp.dot(gxb, wgs_ref[...], preferred_element_type=jnp.float32)
        + jnp.dot(gn_blk, wgn_ref[...], preferred_element_type=jnp.float32)
        + bg_ref[...],
        0.0)

    part = jax.lax.dot_general(
        gxb.astype(jnp.bfloat16), at16,
        (((0,), (0,)), ((), ())),
        preferred_element_type=jnp.float32)

    @pl.when(i == 0)
    def _init():
        acc_ref[...] = part

    @pl.when(i > 0)
    def _accum():
        acc_ref[...] += part

    @pl.when(i == num_blocks - 1)
    def _finish():
        cot_ref[...] = jnp.maximum(
            jnp.dot(wcs_ref[...], cxt_ref[...],
                    preferred_element_type=jnp.float32)
            + jnp.dot(wcn_ref[...], acc_ref[...],
                      preferred_element_type=jnp.float32)
            + bc_ref[...],
            0.0)


def kernel(cell_x, gene_x, adjacency,
           W_cell_self, b_cell_self, W_cell_neigh, b_cell_neigh,
           W_gene_self, b_gene_self, W_gene_neigh, b_gene_neigh):
    N_c, d = cell_x.shape
    N_g = gene_x.shape[0]

    num_blocks = N_g // _BLOCK_G
    assert num_blocks * _BLOCK_G == N_g

    at = adjacency.T
    cxt = cell_x.T
    wgs = W_gene_self.T
    wgn = W_gene_neigh.T
    bc = (b_cell_self + b_cell_neigh).reshape(d, 1)
    bg = (b_gene_self + b_gene_neigh).reshape(1, d)

    full = lambda shape: pl.BlockSpec(shape, lambda i: (0, 0))

    gene_out, cot = pl.pallas_call(
        functools.partial(_body, num_blocks),
        grid=(num_blocks,),
        in_specs=[
            pl.BlockSpec((_BLOCK_G, N_c), lambda i: (i, 0)),
            pl.BlockSpec((_BLOCK_G, d), lambda i: (i, 0)),
            full((d, N_c)),
            full((d, d)), full((d, d)), full((d, 1)),
            full((d, d)), full((d, d)), full((1, d)),
        ],
        out_specs=[
            pl.BlockSpec((_BLOCK_G, d), lambda i: (i, 0)),
            full((d, N_c)),
        ],
        out_shape=[
            jax.ShapeDtypeStruct((N_g, d), jnp.float32),
            jax.ShapeDtypeStruct((d, N_c), jnp.float32),
        ],
        scratch_shapes=[
            pltpu.VMEM((d, N_c), jnp.float32),
            pltpu.VMEM((d, N_c), jnp.bfloat16),
        ],
        compiler_params=pltpu.CompilerParams(
            dimension_semantics=("arbitrary",),
        ),
    )(at, gene_x, cxt,
      W_cell_self, W_cell_neigh, bc, wgs, wgn, bg)

    return (cot.T, gene_out)


# speedup vs baseline: 3.3335x; 1.0157x over previous
import functools

import jax
import jax.numpy as jnp
from jax.experimental import pallas as pl
from jax.experimental.pallas import tpu as pltpu

_BLOCK_G = 200


def _body(num_blocks,
          at_ref, gxb_ref, cxt_ref,
          wcs_ref, wcn_ref, bc_ref,
          wgs_ref, wgn_ref, bg_ref,
          gene_out_ref, cot_ref,
          acc_ref):
    i = pl.program_id(0)

    at = at_ref[...]
    gxb = gxb_ref[...]

    gn_blk = jax.lax.dot_general(
        at, cxt_ref[...], (((1,), (1,)), ((), ())),
        precision=jax.lax.Precision.DEFAULT,
        preferred_element_type=jnp.float32)
    gene_out_ref[...] = jnp.maximum(
        jnp.dot(gxb, wgs_ref[...], preferred_element_type=jnp.float32)
        + jnp.dot(gn_blk, wgn_ref[...], preferred_element_type=jnp.float32)
        + bg_ref[...],
        0.0)

    part = jax.lax.dot_general(
        gxb, at,
        (((0,), (0,)), ((), ())),
        precision=jax.lax.Precision.DEFAULT,
        preferred_element_type=jnp.float32)

    @pl.when(i == 0)
    def _init():
        acc_ref[...] = part

    @pl.when(i > 0)
    def _accum():
        acc_ref[...] += part

    @pl.when(i == num_blocks - 1)
    def _finish():
        cot_ref[...] = jnp.maximum(
            jnp.dot(wcs_ref[...], cxt_ref[...],
                    preferred_element_type=jnp.float32)
            + jnp.dot(wcn_ref[...], acc_ref[...],
                      preferred_element_type=jnp.float32)
            + bc_ref[...],
            0.0)


def kernel(cell_x, gene_x, adjacency,
           W_cell_self, b_cell_self, W_cell_neigh, b_cell_neigh,
           W_gene_self, b_gene_self, W_gene_neigh, b_gene_neigh):
    N_c, d = cell_x.shape
    N_g = gene_x.shape[0]

    num_blocks = N_g // _BLOCK_G
    assert num_blocks * _BLOCK_G == N_g

    at = adjacency.T
    cxt = cell_x.T
    wgs = W_gene_self.T
    wgn = W_gene_neigh.T
    bc = (b_cell_self + b_cell_neigh).reshape(d, 1)
    bg = (b_gene_self + b_gene_neigh).reshape(1, d)

    full = lambda shape: pl.BlockSpec(shape, lambda i: (0, 0))

    gene_out, cot = pl.pallas_call(
        functools.partial(_body, num_blocks),
        grid=(num_blocks,),
        in_specs=[
            pl.BlockSpec((_BLOCK_G, N_c), lambda i: (i, 0)),
            pl.BlockSpec((_BLOCK_G, d), lambda i: (i, 0)),
            full((d, N_c)),
            full((d, d)), full((d, d)), full((d, 1)),
            full((d, d)), full((d, d)), full((1, d)),
        ],
        out_specs=[
            pl.BlockSpec((_BLOCK_G, d), lambda i: (i, 0)),
            full((d, N_c)),
        ],
        out_shape=[
            jax.ShapeDtypeStruct((N_g, d), jnp.float32),
            jax.ShapeDtypeStruct((d, N_c), jnp.float32),
        ],
        scratch_shapes=[
            pltpu.VMEM((d, N_c), jnp.float32),
        ],
        compiler_params=pltpu.CompilerParams(
            dimension_semantics=("arbitrary",),
        ),
    )(at, gene_x, cxt,
      W_cell_self, W_cell_neigh, bc, wgs, wgn, bg)

    return (cot.T, gene_out)
